# baseline (speedup 1.0000x reference)
import functools

import jax
import jax.numpy as jnp
from jax import lax
from jax.experimental import pallas as pl
from jax.experimental.pallas import tpu as pltpu
from jax.experimental.pallas import tpu_sc as plsc

N = 10000
E = 160000
DIN = 128
DE = 16
H = 8
G = 64

NP = 10240
EP = 163840
RW = 144
NW = 32
C = 128
TOTCH_EA = E // C
NBUF = 4
TOTCH_PAD = 16 * 80 * C
RPT = NP // 16

_f32 = jnp.float32


def _tc_a_body(x_ref, w_ref, r_ref, b_ref, z_ref, xr_ref):
    xv = x_ref[...]
    z_ref[...] = lax.dot(xv, w_ref[...], preferred_element_type=_f32)
    xr_ref[...] = lax.dot(xv, r_ref[...], preferred_element_type=_f32) + b_ref[...]


def _tc_b_body(aggp_ref, xr1_ref, w2_ref, r2_ref, b2_ref, z2_ref, xr2_ref):
    agg = aggp_ref[0, :, 0:8] + aggp_ref[1, :, 0:8]
    h1 = jnp.maximum(agg + xr1_ref[...], 0.0)
    z2_ref[...] = lax.dot(h1, w2_ref[...], preferred_element_type=_f32)
    xr2_ref[...] = lax.dot(h1, r2_ref[...], preferred_element_type=_f32) + b2_ref[...]


def _tc_c_body(aggp_ref, xr2_ref, batch_ref, w3_ref, b3_ref, out_ref):
    agg = aggp_ref[0, :, 0:8] + aggp_ref[1, :, 0:8]
    h2 = jnp.maximum(agg + xr2_ref[...], 0.0)
    s = lax.dot(h2, w3_ref[...], preferred_element_type=_f32)
    bt = batch_ref[...]
    gid = lax.broadcasted_iota(jnp.int32, (G, NP), 0)
    m = (gid == bt).astype(_f32)
    out_ref[...] = lax.dot(m, s, preferred_element_type=_f32) + b3_ref[...]


def _vgather(x, idx):
    dnums = lax.GatherDimensionNumbers(
        offset_dims=(), collapsed_slice_dims=(0,), start_index_map=(0,))
    return lax.gather(x, idx[:, None], dnums, (1,),
                      mode=lax.GatherScatterMode.PROMISE_IN_BOUNDS)


_sc_mesh = plsc.VectorSubcoreMesh(core_axis_name="c", subcore_axis_name="s")


def _build_sc_pass(staged, q0, q1):
    qmax = max(q0, q1)
    scratch = [
        pltpu.VMEM((qmax, C), jnp.int32),
        pltpu.VMEM((qmax, C), jnp.int32),
        pltpu.VMEM((NBUF, C, DE), _f32),
        pltpu.VMEM((NBUF, C, RW), _f32),
        pltpu.VMEM((C, 16), _f32),
        pltpu.VMEM_SHARED((NP, 16), _f32),
    ]
    if staged:
        scratch.append(pltpu.VMEM_SHARED((NP, RW), _f32))
    scratch += [pltpu.SemaphoreType.DMA] * (2 * NBUF)

    @functools.partial(
        pl.kernel,
        out_type=jax.ShapeDtypeStruct((2, NP, 16), _f32),
        mesh=_sc_mesh,
        compiler_params=pltpu.CompilerParams(use_tc_tiling_on_sc=False),
        scratch_types=scratch,
    )
    def _sc_pass(table, src3, dst3, ea3, out, srcall, dstall,
                 earing, rowring, msgv, agg, *rest):
        if staged:
            tabsh = rest[0]
            sems = rest[1:]
        else:
            tabsh = table
            sems = rest
        cid = lax.axis_index("c")
        sid = lax.axis_index("s")
        nch = jnp.where(cid == 0, q0, q1)
        chunk0 = jnp.where(cid == 0, sid * q0, 16 * q0 + sid * q1)
        semg = sems[:NBUF]
        seme = sems[NBUF:]

        @pl.when(cid == 0)
        def _():
            pltpu.async_copy(src3.at[pl.ds(chunk0, q0)],
                             srcall.at[pl.ds(0, q0)], semg[0])
            pltpu.async_copy(dst3.at[pl.ds(chunk0, q0)],
                             dstall.at[pl.ds(0, q0)], seme[0])

        @pl.when(cid != 0)
        def _():
            pltpu.async_copy(src3.at[pl.ds(chunk0, q1)],
                             srcall.at[pl.ds(0, q1)], semg[0])
            pltpu.async_copy(dst3.at[pl.ds(chunk0, q1)],
                             dstall.at[pl.ds(0, q1)], seme[0])

        zv = jnp.zeros((16,), _f32)

        def zero_body(r, carry):
            msgv[r, :] = zv
            return carry

        lax.fori_loop(0, C, zero_body, 0)
        for k in range(RPT // C):
            pltpu.sync_copy(msgv, agg.at[pl.ds(sid * RPT + k * C, C)])

        @pl.when(cid == 0)
        def _():
            pltpu.make_async_copy(src3.at[pl.ds(0, q0)],
                                  srcall.at[pl.ds(0, q0)], semg[0]).wait()
            pltpu.make_async_copy(dst3.at[pl.ds(0, q0)],
                                  dstall.at[pl.ds(0, q0)], seme[0]).wait()

        @pl.when(cid != 0)
        def _():
            pltpu.make_async_copy(src3.at[pl.ds(0, q1)],
                                  srcall.at[pl.ds(0, q1)], semg[0]).wait()
            pltpu.make_async_copy(dst3.at[pl.ds(0, q1)],
                                  dstall.at[pl.ds(0, q1)], seme[0]).wait()

        plsc.subcore_barrier()

        with jax.named_scope("scprologue"):
            _noop = 0
        for k in range(NBUF - 1):
            pltpu.async_copy(tabsh.at[srcall.at[k]], rowring.at[k], semg[k])
            pltpu.async_copy(
                ea3.at[pl.ds(jnp.minimum(chunk0 + k, TOTCH_EA - 1) * C, C)],
                earing.at[k], seme[k])

        iota = lax.iota(jnp.int32, 16)
        hi = lax.shift_right_logical(iota, 3)
        foldpat = (iota & 7) + 8

        def quad_body(j, carry):
            for b in range(NBUF):
                i = NBUF * j + b
                pb = (b + NBUF - 1) % NBUF

                @pl.when(i + NBUF - 1 < nch)
                def _():
                    pltpu.async_copy(tabsh.at[srcall.at[i + NBUF - 1]],
                                     rowring.at[pb], semg[pb])
                    g = jnp.minimum(chunk0 + i + NBUF - 1, TOTCH_EA - 1)
                    pltpu.async_copy(ea3.at[pl.ds(g * C, C)],
                                     earing.at[pb], seme[pb])

                pltpu.make_async_copy(table.at[pl.ds(0, C)],
                                      rowring.at[b], semg[b]).wait()
                pltpu.make_async_copy(ea3.at[pl.ds(0, C)],
                                      earing.at[b], seme[b]).wait()

                rowsv = rowring.at[b]
                eav = earing.at[b]

                def edge_body(e4, c2):
                    for u in range(4):
                        e = e4 * 4 + u
                        ea_e = eav[e, :]
                        acc = rowsv[e, pl.ds(8 * 16, 16)]
                        for v in range(8):
                            w = _vgather(ea_e, 2 * v + hi)
                            acc = acc + rowsv[e, pl.ds(v * 16, 16)] * w
                        res = acc + _vgather(acc, foldpat)
                        msgv[e, :] = res
                    return c2

                lax.fori_loop(0, C // 4, edge_body, 0)
                pltpu.sync_copy(msgv, agg.at[dstall.at[i]], add=True)
            return carry

        with jax.named_scope("scloop"):
            lax.fori_loop(0, nch // NBUF, quad_body, 0)

        with jax.named_scope("sctail"):
            plsc.subcore_barrier()
            pltpu.sync_copy(agg.at[pl.ds(sid * RPT, RPT)],
                            out.at[cid, pl.ds(sid * RPT, RPT)])

    return _sc_pass


_sc_pass1 = _build_sc_pass(staged=False, q0=40, q1=40)
_sc_pass2 = _sc_pass1


def _edge_table_weights(nn_W, nn_b, din):
    wr = nn_W.reshape(DE, din, H).transpose(1, 0, 2).reshape(din, DE * H)
    return jnp.concatenate(
        [wr, nn_b.reshape(din, H),
         jnp.zeros((din, RW - DE * H - H), _f32)], axis=1)


def kernel(x, edge_index, edge_attr, batch, nn1_W, nn1_b, root1, b1,
           nn2_W, nn2_b, root2, b2, W3, b3):
    src = edge_index[0]
    dst = edge_index[1]
    pad_e = 16 * 80 * C - E
    pad_iota = lax.iota(jnp.int32, pad_e)
    src_p = jnp.concatenate([src, pad_iota % N]).reshape(16 * 80, C)
    dst_p = jnp.concatenate([dst, N + pad_iota % (NP - N)]).reshape(16 * 80, C)
    ea_p = edge_attr
    x_p = jnp.pad(x, ((0, NP - N), (0, 0)))
    batch_p = jnp.pad(batch, (0, NP - N), constant_values=G).reshape(1, NP)

    w1cat = _edge_table_weights(nn1_W, nn1_b, DIN)
    w2cat = _edge_table_weights(nn2_W, nn2_b, H)
    b1r = b1.reshape(1, H)
    b2r = b2.reshape(1, H)
    w3p = jnp.pad(W3, ((0, 0), (0, 8 - W3.shape[1])))
    b3p = jnp.broadcast_to(b3.reshape(1, 1), (1, 8))

    z1, xr1 = pl.pallas_call(
        _tc_a_body,
        out_shape=[jax.ShapeDtypeStruct((NP, RW), _f32),
                   jax.ShapeDtypeStruct((NP, H), _f32)],
    )(x_p, w1cat, root1, b1r)

    agg1 = _sc_pass1(z1, src_p, dst_p, ea_p)

    z2, xr2 = pl.pallas_call(
        _tc_b_body,
        out_shape=[jax.ShapeDtypeStruct((NP, RW), _f32),
                   jax.ShapeDtypeStruct((NP, H), _f32)],
    )(agg1, xr1, w2cat, root2, b2r)

    agg2 = _sc_pass2(z2, src_p, dst_p, ea_p)

    out8 = pl.pallas_call(
        _tc_c_body,
        out_shape=jax.ShapeDtypeStruct((G, 8), _f32),
    )(agg2, xr2, batch_p, w3p, b3p)

    return out8[:, 0]


# speedup vs baseline: 1.0324x; 1.0324x over previous
import functools

import jax
import jax.numpy as jnp
from jax import lax
from jax.experimental import pallas as pl
from jax.experimental.pallas import tpu as pltpu
from jax.experimental.pallas import tpu_sc as plsc

N = 10000
E = 160000
DIN = 128
DE = 16
H = 8
G = 64

NP = 10240
EP = 163840
RW = 144
NW = 32
C = 128
TOTCH_EA = E // C
NBUF = 4
TOTCH_PAD = 16 * 80 * C
RPT = NP // 16

_f32 = jnp.float32


def _tc_a_body(x_ref, w_ref, r_ref, b_ref, z_ref, xr_ref):
    xv = x_ref[...]
    z_ref[...] = lax.dot(xv, w_ref[...], preferred_element_type=_f32)
    xr_ref[...] = lax.dot(xv, r_ref[...], preferred_element_type=_f32) + b_ref[...]


def _tc_b_body(aggp_ref, xr1_ref, w2_ref, r2_ref, b2_ref, z2_ref, xr2_ref):
    agg = aggp_ref[0, :, 0:8] + aggp_ref[1, :, 0:8]
    h1 = jnp.maximum(agg + xr1_ref[...], 0.0)
    z2_ref[...] = lax.dot(h1, w2_ref[...], preferred_element_type=_f32)
    xr2_ref[...] = lax.dot(h1, r2_ref[...], preferred_element_type=_f32) + b2_ref[...]


def _tc_c_body(aggp_ref, xr2_ref, batch_ref, w3_ref, b3_ref, out_ref):
    agg = aggp_ref[0, :, 0:8] + aggp_ref[1, :, 0:8]
    h2 = jnp.maximum(agg + xr2_ref[...], 0.0)
    s = lax.dot(h2, w3_ref[...], preferred_element_type=_f32)
    bt = batch_ref[...]
    gid = lax.broadcasted_iota(jnp.int32, (G, NP), 0)
    m = (gid == bt).astype(_f32)
    out_ref[...] = lax.dot(m, s, preferred_element_type=_f32) + b3_ref[...]


def _vgather(x, idx):
    dnums = lax.GatherDimensionNumbers(
        offset_dims=(), collapsed_slice_dims=(0,), start_index_map=(0,))
    return lax.gather(x, idx[:, None], dnums, (1,),
                      mode=lax.GatherScatterMode.PROMISE_IN_BOUNDS)


_sc_mesh = plsc.VectorSubcoreMesh(core_axis_name="c", subcore_axis_name="s")


def _build_sc_pass(staged, q0, q1):
    qmax = max(q0, q1)
    scratch = [
        pltpu.VMEM((qmax, C), jnp.int32),
        pltpu.VMEM((qmax, C), jnp.int32),
        pltpu.VMEM((NBUF, C, DE), _f32),
        pltpu.VMEM((NBUF, C, RW), _f32),
        pltpu.VMEM((C, 16), _f32),
        pltpu.VMEM_SHARED((NP, 16), _f32),
    ]
    if staged:
        scratch.append(pltpu.VMEM_SHARED((NP, RW), _f32))
    scratch += [pltpu.SemaphoreType.DMA] * (2 * NBUF)

    @functools.partial(
        pl.kernel,
        out_type=jax.ShapeDtypeStruct((2, NP, 16), _f32),
        mesh=_sc_mesh,
        compiler_params=pltpu.CompilerParams(use_tc_tiling_on_sc=False),
        scratch_types=scratch,
    )
    def _sc_pass(table, src3, dst3, ea3, out, srcall, dstall,
                 earing, rowring, msgv, agg, *rest):
        if staged:
            tabsh = rest[0]
            sems = rest[1:]
        else:
            tabsh = table
            sems = rest
        cid = lax.axis_index("c")
        sid = lax.axis_index("s")
        nch = jnp.where(cid == 0, q0, q1)
        chunk0 = jnp.where(cid == 0, sid * q0, 16 * q0 + sid * q1)
        semg = sems[:NBUF]
        seme = sems[NBUF:]

        @pl.when(cid == 0)
        def _():
            pltpu.async_copy(src3.at[pl.ds(chunk0, q0)],
                             srcall.at[pl.ds(0, q0)], semg[0])
            pltpu.async_copy(dst3.at[pl.ds(chunk0, q0)],
                             dstall.at[pl.ds(0, q0)], seme[0])

        @pl.when(cid != 0)
        def _():
            pltpu.async_copy(src3.at[pl.ds(chunk0, q1)],
                             srcall.at[pl.ds(0, q1)], semg[0])
            pltpu.async_copy(dst3.at[pl.ds(chunk0, q1)],
                             dstall.at[pl.ds(0, q1)], seme[0])

        zv = jnp.zeros((16,), _f32)

        def zero_body(r, carry):
            msgv[r, :] = zv
            return carry

        lax.fori_loop(0, C, zero_body, 0)
        for k in range(RPT // C):
            pltpu.sync_copy(msgv, agg.at[pl.ds(sid * RPT + k * C, C)])

        @pl.when(cid == 0)
        def _():
            pltpu.make_async_copy(src3.at[pl.ds(0, q0)],
                                  srcall.at[pl.ds(0, q0)], semg[0]).wait()
            pltpu.make_async_copy(dst3.at[pl.ds(0, q0)],
                                  dstall.at[pl.ds(0, q0)], seme[0]).wait()

        @pl.when(cid != 0)
        def _():
            pltpu.make_async_copy(src3.at[pl.ds(0, q1)],
                                  srcall.at[pl.ds(0, q1)], semg[0]).wait()
            pltpu.make_async_copy(dst3.at[pl.ds(0, q1)],
                                  dstall.at[pl.ds(0, q1)], seme[0]).wait()

        plsc.subcore_barrier()

        with jax.named_scope("scprologue"):
            _noop = 0
        for k in range(NBUF - 1):
            pltpu.async_copy(tabsh.at[srcall.at[k]], rowring.at[k], semg[k])
            pltpu.async_copy(
                ea3.at[pl.ds(jnp.minimum(chunk0 + k, TOTCH_EA - 1) * C, C)],
                earing.at[k], seme[k])

        iota = lax.iota(jnp.int32, 16)
        hi = lax.shift_right_logical(iota, 3)
        foldpat = (iota & 7) + 8

        def quad_body(j, carry):
            for b in range(NBUF):
                i = NBUF * j + b
                pb = (b + NBUF - 1) % NBUF

                @pl.when(i + NBUF - 1 < nch)
                def _():
                    pltpu.async_copy(tabsh.at[srcall.at[i + NBUF - 1]],
                                     rowring.at[pb], semg[pb])
                    g = jnp.minimum(chunk0 + i + NBUF - 1, TOTCH_EA - 1)
                    pltpu.async_copy(ea3.at[pl.ds(g * C, C)],
                                     earing.at[pb], seme[pb])

                pltpu.make_async_copy(table.at[pl.ds(0, C)],
                                      rowring.at[b], semg[b]).wait()
                pltpu.make_async_copy(ea3.at[pl.ds(0, C)],
                                      earing.at[b], seme[b]).wait()

                rowsv = rowring.at[b]
                eav = earing.at[b]

                def edge_body(e4, c2):
                    for u in range(4):
                        e = e4 * 4 + u
                        ea_e = eav[e, :]
                        ps = [rowsv[e, pl.ds(8 * 16, 16)]]
                        for v in range(8):
                            w = _vgather(ea_e, 2 * v + hi)
                            ps.append(rowsv[e, pl.ds(v * 16, 16)] * w)
                        while len(ps) > 1:
                            ps = [a + b for a, b in zip(ps[::2], ps[1::2])] + \
                                 (ps[-1:] if len(ps) % 2 else [])
                        acc = ps[0]
                        res = acc + _vgather(acc, foldpat)
                        msgv[e, :] = res
                    return c2

                lax.fori_loop(0, C // 4, edge_body, 0)
                pltpu.sync_copy(msgv, agg.at[dstall.at[i]], add=True)
            return carry

        with jax.named_scope("scloop"):
            lax.fori_loop(0, nch // NBUF, quad_body, 0)

        with jax.named_scope("sctail"):
            plsc.subcore_barrier()
            pltpu.sync_copy(agg.at[pl.ds(sid * RPT, RPT)],
                            out.at[cid, pl.ds(sid * RPT, RPT)])

    return _sc_pass


_sc_pass1 = _build_sc_pass(staged=False, q0=40, q1=40)
_sc_pass2 = _sc_pass1


def _edge_table_weights(nn_W, nn_b, din):
    wr = nn_W.reshape(DE, din, H).transpose(1, 0, 2).reshape(din, DE * H)
    return jnp.concatenate(
        [wr, nn_b.reshape(din, H),
         jnp.zeros((din, RW - DE * H - H), _f32)], axis=1)


def kernel(x, edge_index, edge_attr, batch, nn1_W, nn1_b, root1, b1,
           nn2_W, nn2_b, root2, b2, W3, b3):
    src = edge_index[0]
    dst = edge_index[1]
    pad_e = 16 * 80 * C - E
    pad_iota = lax.iota(jnp.int32, pad_e)
    src_p = jnp.concatenate([src, pad_iota % N]).reshape(16 * 80, C)
    dst_p = jnp.concatenate([dst, N + pad_iota % (NP - N)]).reshape(16 * 80, C)
    ea_p = edge_attr
    x_p = jnp.pad(x, ((0, NP - N), (0, 0)))
    batch_p = jnp.pad(batch, (0, NP - N), constant_values=G).reshape(1, NP)

    w1cat = _edge_table_weights(nn1_W, nn1_b, DIN)
    w2cat = _edge_table_weights(nn2_W, nn2_b, H)
    b1r = b1.reshape(1, H)
    b2r = b2.reshape(1, H)
    w3p = jnp.pad(W3, ((0, 0), (0, 8 - W3.shape[1])))
    b3p = jnp.broadcast_to(b3.reshape(1, 1), (1, 8))

    z1, xr1 = pl.pallas_call(
        _tc_a_body,
        out_shape=[jax.ShapeDtypeStruct((NP, RW), _f32),
                   jax.ShapeDtypeStruct((NP, H), _f32)],
    )(x_p, w1cat, root1, b1r)

    agg1 = _sc_pass1(z1, src_p, dst_p, ea_p)

    z2, xr2 = pl.pallas_call(
        _tc_b_body,
        out_shape=[jax.ShapeDtypeStruct((NP, RW), _f32),
                   jax.ShapeDtypeStruct((NP, H), _f32)],
    )(agg1, xr1, w2cat, root2, b2r)

    agg2 = _sc_pass2(z2, src_p, dst_p, ea_p)

    out8 = pl.pallas_call(
        _tc_c_body,
        out_shape=jax.ShapeDtypeStruct((G, 8), _f32),
    )(agg2, xr2, batch_p, w3p, b3p)

    return out8[:, 0]
